# baseline (speedup 1.0000x reference)
import functools

import jax
import jax.numpy as jnp
from jax import lax
from jax.experimental import pallas as pl
from jax.experimental.pallas import tpu as pltpu
from jax.experimental.pallas import tpu_sc as plsc

_BATCH = 1024
_TGT = 50
_VOCAB = 100000
_NC = 2
_NS = 16
_L = 16
_NW = _NC * _NS
_ROWS_W = _BATCH // _NW
_E = _ROWS_W * _TGT
_NVEC = _E // _L
_CH = 128
_NFULL = _E // _CH
_REM = _E - _NFULL * _CH


def _make_sc_call():
    mesh = plsc.VectorSubcoreMesh(
        core_axis_name="c", subcore_axis_name="s",
        num_cores=_NC, num_subcores=_NS)

    @functools.partial(
        pl.kernel,
        out_type=jax.ShapeDtypeStruct((_NW, _L), jnp.float32),
        mesh=mesh,
        scratch_types=[
            pltpu.VMEM((_E,), jnp.int32),
            pltpu.VMEM((_E,), jnp.int32),
            pltpu.VMEM((_E,), jnp.float32),
            pltpu.VMEM((_E,), jnp.float32),
            pltpu.VMEM((_L,), jnp.float32),
            pltpu.SemaphoreType.DMA,
        ],
    )
    def bow_partials(out_hbm, tgt_hbm, wt_hbm, part_hbm,
                     tgt_v, idx_v, val_v, wv_v, acc_v, sem):
        cid = lax.axis_index("c")
        sid = lax.axis_index("s")
        wid = sid * _NC + cid
        base_elem = wid * _E
        base_row = wid * _ROWS_W

        pltpu.sync_copy(tgt_hbm.at[pl.ds(base_elem, _E)], tgt_v)

        def idx_body(c, carry):
            off = c * _L
            t = tgt_v[pl.ds(off, _L)]
            pos = off + lax.iota(jnp.int32, _L)
            row = base_row + lax.div(pos, _TGT)
            idx_v[pl.ds(off, _L)] = row * _VOCAB + t
            return carry
        lax.fori_loop(0, _NVEC, idx_body, 0)

        copies = []
        for c in range(_NFULL + 1):
            off = c * _CH
            sz = _CH if c < _NFULL else _REM
            copies.append(pltpu.async_copy(
                out_hbm.at[idx_v.at[pl.ds(off, sz)]],
                val_v.at[pl.ds(off, sz)], sem))
            copies.append(pltpu.async_copy(
                wt_hbm.at[tgt_v.at[pl.ds(off, sz)]],
                wv_v.at[pl.ds(off, sz)], sem))
        for cp in copies:
            cp.wait()

        def acc_body(c, acc):
            off = c * _L
            return acc + val_v[pl.ds(off, _L)] * wv_v[pl.ds(off, _L)]
        acc = lax.fori_loop(0, _NVEC, acc_body,
                            jnp.zeros((_L,), jnp.float32))
        acc_v[...] = acc
        pltpu.sync_copy(acc_v, part_hbm.at[wid])

    return bow_partials


_bow_partials = _make_sc_call()


def kernel(output, target, weight):
    out_flat = output.reshape(-1)
    tgt_flat = target.reshape(-1)
    partials = _bow_partials(out_flat, tgt_flat, weight)
    return -(jnp.sum(partials) / _BATCH)


# speedup vs baseline: 2.1363x; 2.1363x over previous
import functools

import jax
import jax.numpy as jnp
from jax import lax
from jax.experimental import pallas as pl
from jax.experimental.pallas import tpu as pltpu
from jax.experimental.pallas import tpu_sc as plsc

_BATCH = 1024
_TGT = 50
_VOCAB = 100000
_NC = 2
_NS = 16
_L = 16
_NW = _NC * _NS
_ROWS_W = _BATCH // _NW
_E = _ROWS_W * _TGT
_RUN = 128
_CH = 160
_NCHUNK = _E // _CH
_WCH = 128
_NWCH = _E // _WCH
_WREM = _E - _NWCH * _WCH


def _make_sc_call():
    mesh = plsc.VectorSubcoreMesh(
        core_axis_name="c", subcore_axis_name="s",
        num_cores=_NC, num_subcores=_NS)

    @functools.partial(
        pl.kernel,
        out_type=jax.ShapeDtypeStruct((_NW, _L), jnp.float32),
        mesh=mesh,
        compiler_params=pltpu.CompilerParams(needs_layout_passes=False),
        scratch_types=[
            pltpu.VMEM((_E,), jnp.int32),
            pltpu.VMEM((_CH, _RUN), jnp.float32),
            pltpu.VMEM((_E,), jnp.float32),
            pltpu.VMEM((_L,), jnp.float32),
            pltpu.SemaphoreType.DMA,
            pltpu.SemaphoreType.DMA,
        ],
    )
    def bow_partials(out_hbm, tgt_hbm, wt_hbm, part_hbm,
                     tgt_v, run_v, wv_v, acc_v, sem, wsem):
        cid = lax.axis_index("c")
        sid = lax.axis_index("s")
        wid = sid * _NC + cid
        base_elem = wid * _E
        base_row = wid * _ROWS_W
        lane = lax.iota(jnp.int32, _L)

        pltpu.sync_copy(tgt_hbm.at[pl.ds(base_elem, _E)], tgt_v)

        wcopies = []
        for c in range(_NWCH + 1):
            off = c * _WCH
            sz = _WCH if c < _NWCH else _WREM
            wcopies.append(pltpu.async_copy(
                wt_hbm.at[tgt_v.at[pl.ds(off, sz)]],
                wv_v.at[pl.ds(off, sz)], wsem))

        def chunk_body(g, acc_outer):
            e0 = g * _CH

            def fire_body(i, carry):
                e = e0 + i
                chunk16 = tgt_v[pl.ds((e >> 4) << 4, _L)]
                t = lax.reduce_sum(
                    lax.select(lane == (e & 15), chunk16,
                               jnp.zeros((_L,), jnp.int32)), axes=(0,))
                b = base_row + lax.div(e, _TGT)
                tc = pl.multiple_of((t >> 7) << 7, _RUN)
                pltpu.async_copy(out_hbm.at[b].at[pl.ds(tc, _RUN)],
                                 run_v.at[i], sem)
                return carry
            lax.fori_loop(0, _CH, fire_body, 0)

            def drain_body(i, carry):
                pltpu.make_async_copy(out_hbm.at[0].at[pl.ds(0, _RUN)],
                                      run_v.at[i], sem).wait()
                return carry
            lax.fori_loop(0, _CH, drain_body, 0)

            def acc_body(k, acc):
                off = e0 + k * _L
                t16 = tgt_v[pl.ds(off, _L)]
                v = plsc.load_gather(run_v, [k * _L + lane, t16 & 127])
                return acc + v * wv_v[pl.ds(off, _L)]
            return lax.fori_loop(0, _CH // _L, acc_body, acc_outer)

        for c, cp in enumerate(wcopies):
            cp.wait()

        acc = lax.fori_loop(0, _NCHUNK, chunk_body,
                            jnp.zeros((_L,), jnp.float32))
        acc_v[...] = acc
        pltpu.sync_copy(acc_v, part_hbm.at[wid])

    return bow_partials


_bow_partials = _make_sc_call()


def kernel(output, target, weight):
    tgt_flat = target.reshape(-1)
    partials = _bow_partials(output, tgt_flat, weight)
    return -(jnp.sum(partials) / _BATCH)


# speedup vs baseline: 2.2415x; 1.0493x over previous
import functools

import jax
import jax.numpy as jnp
from jax import lax
from jax.experimental import pallas as pl
from jax.experimental.pallas import tpu as pltpu
from jax.experimental.pallas import tpu_sc as plsc

_BATCH = 1024
_TGT = 50
_VOCAB = 100000
_NC = 2
_NS = 16
_L = 16
_NW = _NC * _NS
_ROWS_W = _BATCH // _NW
_E = _ROWS_W * _TGT
_RUN = 128
_CH = 160
_NCHUNK = _E // _CH
_WCH = 128
_NWCH = _E // _WCH
_WREM = _E - _NWCH * _WCH


def _make_sc_call():
    mesh = plsc.VectorSubcoreMesh(
        core_axis_name="c", subcore_axis_name="s",
        num_cores=_NC, num_subcores=_NS)

    @functools.partial(
        pl.kernel,
        out_type=jax.ShapeDtypeStruct((_NW, _L), jnp.float32),
        mesh=mesh,
        compiler_params=pltpu.CompilerParams(needs_layout_passes=False),
        scratch_types=[
            pltpu.VMEM((_E,), jnp.int32),
            pltpu.VMEM((_CH, _RUN), jnp.float32),
            pltpu.VMEM((_E,), jnp.float32),
            pltpu.VMEM((_L,), jnp.float32),
            pltpu.SemaphoreType.DMA,
            pltpu.SemaphoreType.DMA,
        ],
    )
    def bow_partials(out_hbm, tgt_hbm, wt_hbm, part_hbm,
                     tgt_v, run_v, wv_v, acc_v, sem, wsem):
        cid = lax.axis_index("c")
        sid = lax.axis_index("s")
        wid = sid * _NC + cid
        base_elem = wid * _E
        base_row = wid * _ROWS_W
        lane = lax.iota(jnp.int32, _L)

        pltpu.sync_copy(tgt_hbm.at[pl.ds(base_elem, _E)], tgt_v)

        wcopies = []
        for c in range(_NWCH + 1):
            off = c * _WCH
            sz = _WCH if c < _NWCH else _WREM
            wcopies.append(pltpu.async_copy(
                wt_hbm.at[tgt_v.at[pl.ds(off, sz)]],
                wv_v.at[pl.ds(off, sz)], wsem))

        izeros = jnp.zeros((_L,), jnp.int32)

        def chunk_body(g, acc_outer):
            e0 = g * _CH

            def fire_body(i, carry):
                eb = e0 + i * _L
                chunk16 = tgt_v[pl.ds(eb, _L)]
                for k in range(_L):
                    t = lax.reduce_sum(
                        lax.select(lane == k, chunk16, izeros), axes=(0,))
                    b = base_row + lax.div(eb + k, _TGT)
                    tc = pl.multiple_of((t >> 7) << 7, _RUN)
                    pltpu.async_copy(out_hbm.at[b].at[pl.ds(tc, _RUN)],
                                     run_v.at[i * _L + k], sem)
                return carry
            lax.fori_loop(0, _CH // _L, fire_body, 0)

            pltpu.make_async_copy(
                out_hbm.at[pl.ds(0, _CH)].at[:, pl.ds(0, _RUN)],
                run_v, sem).wait()

            def acc_body(k, acc):
                off = e0 + k * _L
                t16 = tgt_v[pl.ds(off, _L)]
                v = plsc.load_gather(run_v, [k * _L + lane, t16 & 127])
                return acc + v * wv_v[pl.ds(off, _L)]
            return lax.fori_loop(0, _CH // _L, acc_body, acc_outer)

        for c, cp in enumerate(wcopies):
            cp.wait()

        acc = lax.fori_loop(0, _NCHUNK, chunk_body,
                            jnp.zeros((_L,), jnp.float32))
        acc_v[...] = acc
        pltpu.sync_copy(acc_v, part_hbm.at[wid])

    return bow_partials


_bow_partials = _make_sc_call()


def kernel(output, target, weight):
    tgt_flat = target.reshape(-1)
    partials = _bow_partials(output, tgt_flat, weight)
    return -(jnp.sum(partials) / _BATCH)


# speedup vs baseline: 20.6566x; 9.2153x over previous
import functools

import jax
import jax.numpy as jnp
from jax import lax
from jax.experimental import pallas as pl
from jax.experimental.pallas import tpu as pltpu
from jax.experimental.pallas import tpu_sc as plsc

_BATCH = 1024
_TGT = 50
_VOCAB = 100000
_NC = 2
_NS = 16
_L = 16
_NW = _NC * _NS
_ROWS_W = _BATCH // _NW
_E = _ROWS_W * _TGT
_RUN = 128
_CH = 160
_NCHUNK = _E // _CH
_WCH = 128
_NWCH = _E // _WCH
_WREM = _E - _NWCH * _WCH


def _make_sc_call():
    mesh = plsc.VectorSubcoreMesh(
        core_axis_name="c", subcore_axis_name="s",
        num_cores=_NC, num_subcores=_NS)

    @functools.partial(
        pl.kernel,
        out_type=jax.ShapeDtypeStruct((_NW, _L), jnp.float32),
        mesh=mesh,
        compiler_params=pltpu.CompilerParams(needs_layout_passes=False),
        scratch_types=[
            pltpu.VMEM((_E,), jnp.int32),
            pltpu.VMEM((_CH, _RUN), jnp.float32),
            pltpu.VMEM((_E,), jnp.float32),
            pltpu.VMEM((_L,), jnp.float32),
            pltpu.SemaphoreType.DMA,
            pltpu.SemaphoreType.DMA,
        ],
    )
    def bow_partials(outt_hbm, tgt_hbm, wt_hbm, part_hbm,
                     tgt_v, run_v, wv_v, acc_v, sem, wsem):
        cid = lax.axis_index("c")
        sid = lax.axis_index("s")
        wid = sid * _NC + cid
        base_elem = wid * _E
        base_row = wid * _ROWS_W
        rb = pl.multiple_of((base_row >> 7) << 7, _RUN)
        lane = lax.iota(jnp.int32, _L)

        pltpu.sync_copy(tgt_hbm.at[pl.ds(base_elem, _E)], tgt_v)

        wcopies = []
        for c in range(_NWCH + 1):
            off = c * _WCH
            sz = _WCH if c < _NWCH else _WREM
            wcopies.append(pltpu.async_copy(
                wt_hbm.at[tgt_v.at[pl.ds(off, sz)]],
                wv_v.at[pl.ds(off, sz)], wsem))

        izeros = jnp.zeros((_L,), jnp.int32)

        def chunk_body(g, acc_outer):
            e0 = g * _CH

            def fire_body(i, carry):
                eb = e0 + i * _L
                chunk16 = tgt_v[pl.ds(eb, _L)]
                for k in range(_L):
                    t = lax.reduce_sum(
                        lax.select(lane == k, chunk16, izeros), axes=(0,))
                    pltpu.async_copy(outt_hbm.at[t].at[pl.ds(rb, _RUN)],
                                     run_v.at[i * _L + k], sem)
                return carry
            lax.fori_loop(0, _CH // _L, fire_body, 0)

            pltpu.make_async_copy(
                outt_hbm.at[pl.ds(0, _CH)].at[:, pl.ds(0, _RUN)],
                run_v, sem).wait()

            def acc_body(k, acc):
                off = e0 + k * _L
                pos = off + lane
                b = base_row + lax.div(pos, _TGT)
                v = plsc.load_gather(run_v, [k * _L + lane, b & 127])
                return acc + v * wv_v[pl.ds(off, _L)]
            return lax.fori_loop(0, _CH // _L, acc_body, acc_outer)

        for cp in wcopies:
            cp.wait()

        acc = lax.fori_loop(0, _NCHUNK, chunk_body,
                            jnp.zeros((_L,), jnp.float32))
        acc_v[...] = acc
        pltpu.sync_copy(acc_v, part_hbm.at[wid])

    return bow_partials


_bow_partials = _make_sc_call()


def kernel(output, target, weight):
    tgt_flat = target.reshape(-1)
    partials = _bow_partials(output.T, tgt_flat, weight)
    return -(jnp.sum(partials) / _BATCH)


# speedup vs baseline: 21.5195x; 1.0418x over previous
import functools

import jax
import jax.numpy as jnp
from jax import lax
from jax.experimental import pallas as pl
from jax.experimental.pallas import tpu as pltpu
from jax.experimental.pallas import tpu_sc as plsc

_BATCH = 1024
_TGT = 50
_VOCAB = 100000
_NC = 2
_NS = 16
_L = 16
_NW = _NC * _NS
_ROWS_W = _BATCH // _NW
_E = _ROWS_W * _TGT
_RUN = 128
_CH = 160
_NCHUNK = _E // _CH
_WCH = 128
_NWCH = _E // _WCH
_WREM = _E - _NWCH * _WCH


def _make_sc_call():
    mesh = plsc.VectorSubcoreMesh(
        core_axis_name="c", subcore_axis_name="s",
        num_cores=_NC, num_subcores=_NS)

    @functools.partial(
        pl.kernel,
        out_type=jax.ShapeDtypeStruct((_NW, _L), jnp.float32),
        mesh=mesh,
        compiler_params=pltpu.CompilerParams(needs_layout_passes=False),
        scratch_types=[
            pltpu.VMEM((_E,), jnp.int32),
            pltpu.VMEM((_CH, _RUN), jnp.float32),
            pltpu.VMEM((_CH, _RUN), jnp.float32),
            pltpu.VMEM((_E,), jnp.float32),
            pltpu.VMEM((_L,), jnp.float32),
            pltpu.SemaphoreType.DMA,
            pltpu.SemaphoreType.DMA,
            pltpu.SemaphoreType.DMA,
        ],
    )
    def bow_partials(outt_hbm, tgt_hbm, wt_hbm, part_hbm,
                     tgt_v, run_a, run_b, wv_v, acc_v, sem_a, sem_b, wsem):
        cid = lax.axis_index("c")
        sid = lax.axis_index("s")
        wid = sid * _NC + cid
        base_elem = wid * _E
        base_row = wid * _ROWS_W
        rb = pl.multiple_of((base_row >> 7) << 7, _RUN)
        lane = lax.iota(jnp.int32, _L)

        pltpu.sync_copy(tgt_hbm.at[pl.ds(base_elem, _E)], tgt_v)

        wcopies = []
        for c in range(_NWCH + 1):
            off = c * _WCH
            sz = _WCH if c < _NWCH else _WREM
            wcopies.append(pltpu.async_copy(
                wt_hbm.at[tgt_v.at[pl.ds(off, sz)]],
                wv_v.at[pl.ds(off, sz)], wsem))

        izeros = jnp.zeros((_L,), jnp.int32)
        bufs = [run_a, run_b]
        sems = [sem_a, sem_b]

        def fire(g):
            e0 = g * _CH
            run_v, sem = bufs[g % 2], sems[g % 2]

            def fire_body(i, carry):
                eb = e0 + i * _L
                chunk16 = tgt_v[pl.ds(eb, _L)]
                for k in range(_L):
                    t = lax.reduce_sum(
                        lax.select(lane == k, chunk16, izeros), axes=(0,))
                    pltpu.async_copy(outt_hbm.at[t].at[pl.ds(rb, _RUN)],
                                     run_v.at[i * _L + k], sem)
                return carry
            lax.fori_loop(0, _CH // _L, fire_body, 0)

        def drain_and_acc(g, acc_outer):
            e0 = g * _CH
            run_v, sem = bufs[g % 2], sems[g % 2]
            pltpu.make_async_copy(
                outt_hbm.at[pl.ds(0, _CH)].at[:, pl.ds(0, _RUN)],
                run_v, sem).wait()

            def acc_body(k, acc):
                off = e0 + k * _L
                pos = off + lane
                b = base_row + lax.div(pos, _TGT)
                v = plsc.load_gather(run_v, [k * _L + lane, b & 127])
                return acc + v * wv_v[pl.ds(off, _L)]
            return lax.fori_loop(0, _CH // _L, acc_body, acc_outer)

        for cp in wcopies:
            cp.wait()

        acc = jnp.zeros((_L,), jnp.float32)
        fire(0)
        for g in range(1, _NCHUNK):
            fire(g)
            acc = drain_and_acc(g - 1, acc)
        acc = drain_and_acc(_NCHUNK - 1, acc)
        acc_v[...] = acc
        pltpu.sync_copy(acc_v, part_hbm.at[wid])

    return bow_partials


_bow_partials = _make_sc_call()


def kernel(output, target, weight):
    tgt_flat = target.reshape(-1)
    partials = _bow_partials(output.T, tgt_flat, weight)
    return -(jnp.sum(partials) / _BATCH)


# speedup vs baseline: 21.7162x; 1.0091x over previous
import functools

import jax
import jax.numpy as jnp
from jax import lax
from jax.experimental import pallas as pl
from jax.experimental.pallas import tpu as pltpu
from jax.experimental.pallas import tpu_sc as plsc

_BATCH = 1024
_TGT = 50
_VOCAB = 100000
_NC = 2
_NS = 16
_L = 16
_NW = _NC * _NS
_ROWS_W = _BATCH // _NW
_E = _ROWS_W * _TGT
_RUN = 128
_CH = 160
_NCHUNK = _E // _CH
_WCH = 128
_NWCH = _E // _WCH
_WREM = _E - _NWCH * _WCH


def _make_sc_call():
    mesh = plsc.VectorSubcoreMesh(
        core_axis_name="c", subcore_axis_name="s",
        num_cores=_NC, num_subcores=_NS)

    @functools.partial(
        pl.kernel,
        out_type=jax.ShapeDtypeStruct((_NW, _L), jnp.float32),
        mesh=mesh,
        compiler_params=pltpu.CompilerParams(needs_layout_passes=False),
        scratch_types=[
            pltpu.VMEM((_E,), jnp.int32),
            pltpu.VMEM((_CH, _RUN), jnp.float32),
            pltpu.VMEM((_CH, _RUN), jnp.float32),
            pltpu.VMEM((_E,), jnp.float32),
            pltpu.VMEM((_L,), jnp.float32),
            pltpu.SemaphoreType.DMA,
            pltpu.SemaphoreType.DMA,
            pltpu.SemaphoreType.DMA,
        ],
    )
    def bow_partials(outt_hbm, tgt_hbm, wt_hbm, part_hbm,
                     tgt_v, run_a, run_b, wv_v, acc_v, sem_a, sem_b, wsem):
        cid = lax.axis_index("c")
        sid = lax.axis_index("s")
        wid = sid * _NC + cid
        base_elem = wid * _E
        base_row = wid * _ROWS_W
        rb = pl.multiple_of((base_row >> 7) << 7, _RUN)
        lane = lax.iota(jnp.int32, _L)

        pltpu.sync_copy(tgt_hbm.at[pl.ds(base_elem, _E)], tgt_v)

        wcopies = []
        for c in range(_NWCH + 1):
            off = c * _WCH
            sz = _WCH if c < _NWCH else _WREM
            wcopies.append(pltpu.async_copy(
                wt_hbm.at[tgt_v.at[pl.ds(off, sz)]],
                wv_v.at[pl.ds(off, sz)], wsem))

        izeros = jnp.zeros((_L,), jnp.int32)
        bufs = [run_a, run_b]
        sems = [sem_a, sem_b]

        def fire(g):
            e0 = g * _CH
            run_v, sem = bufs[g % 2], sems[g % 2]

            def fire_body(i, carry):
                eb = e0 + i * _L
                chunk16 = tgt_v[pl.ds(eb, _L)]
                for k in range(_L):
                    t = chunk16[k]
                    pltpu.async_copy(outt_hbm.at[t].at[pl.ds(rb, _RUN)],
                                     run_v.at[i * _L + k], sem)
                return carry
            lax.fori_loop(0, _CH // _L, fire_body, 0)

        def drain_and_acc(g, acc_outer):
            e0 = g * _CH
            run_v, sem = bufs[g % 2], sems[g % 2]
            pltpu.make_async_copy(
                outt_hbm.at[pl.ds(0, _CH)].at[:, pl.ds(0, _RUN)],
                run_v, sem).wait()

            def acc_body(k, acc):
                off = e0 + k * _L
                pos = off + lane
                b = base_row + lax.div(pos, _TGT)
                v = plsc.load_gather(run_v, [k * _L + lane, b & 127])
                return acc + v * wv_v[pl.ds(off, _L)]
            return lax.fori_loop(0, _CH // _L, acc_body, acc_outer)

        for cp in wcopies:
            cp.wait()

        acc = jnp.zeros((_L,), jnp.float32)
        fire(0)
        for g in range(1, _NCHUNK):
            fire(g)
            acc = drain_and_acc(g - 1, acc)
        acc = drain_and_acc(_NCHUNK - 1, acc)
        acc_v[...] = acc
        pltpu.sync_copy(acc_v, part_hbm.at[wid])

    return bow_partials


_bow_partials = _make_sc_call()


def kernel(output, target, weight):
    tgt_flat = target.reshape(-1)
    partials = _bow_partials(output.T, tgt_flat, weight)
    return -(jnp.sum(partials) / _BATCH)


# speedup vs baseline: 21.7666x; 1.0023x over previous
import functools

import jax
import jax.numpy as jnp
from jax import lax
from jax.experimental import pallas as pl
from jax.experimental.pallas import tpu as pltpu
from jax.experimental.pallas import tpu_sc as plsc

_BATCH = 1024
_TGT = 50
_VOCAB = 100000
_NC = 2
_NS = 16
_L = 16
_NW = _NC * _NS
_ROWS_W = _BATCH // _NW
_E = _ROWS_W * _TGT
_RUN = 128
_CH = 160
_NCHUNK = _E // _CH
_WCH = 128
_NWCH = _E // _WCH
_WREM = _E - _NWCH * _WCH


def _make_sc_call():
    mesh = plsc.VectorSubcoreMesh(
        core_axis_name="c", subcore_axis_name="s",
        num_cores=_NC, num_subcores=_NS)

    @functools.partial(
        pl.kernel,
        out_type=jax.ShapeDtypeStruct((_NW, _L), jnp.float32),
        mesh=mesh,
        compiler_params=pltpu.CompilerParams(needs_layout_passes=False),
        scratch_types=[
            pltpu.VMEM((_E,), jnp.int32),
            pltpu.VMEM((_CH, _RUN), jnp.float32),
            pltpu.VMEM((_CH, _RUN), jnp.float32),
            pltpu.VMEM((_E,), jnp.float32),
            pltpu.VMEM((_L,), jnp.float32),
            pltpu.SemaphoreType.DMA,
            pltpu.SemaphoreType.DMA,
            pltpu.SemaphoreType.DMA,
        ],
    )
    def bow_partials(outt_hbm, tgt_hbm, wt_hbm, part_hbm,
                     tgt_v, run_a, run_b, wv_v, acc_v, sem_a, sem_b, wsem):
        cid = lax.axis_index("c")
        sid = lax.axis_index("s")
        wid = sid * _NC + cid
        base_elem = wid * _E
        lane = lax.iota(jnp.int32, _L)

        pltpu.sync_copy(tgt_hbm.at[pl.ds(base_elem, _E)], tgt_v)

        wcopies = []
        for c in range(_NWCH + 1):
            off = c * _WCH
            sz = _WCH if c < _NWCH else _WREM
            wcopies.append(pltpu.async_copy(
                wt_hbm.at[tgt_v.at[pl.ds(off, sz)]],
                wv_v.at[pl.ds(off, sz)], wsem))

        izeros = jnp.zeros((_L,), jnp.int32)
        bufs = [run_a, run_b]
        sems = [sem_a, sem_b]

        def fire(g):
            e0 = g * _CH
            run_v, sem = bufs[g % 2], sems[g % 2]

            def fire_body(i, carry):
                eb = e0 + i * _L
                chunk16 = tgt_v[pl.ds(eb, _L)]
                for k in range(_L):
                    t = chunk16[k]
                    rb = pl.multiple_of(
                        (((base_elem + eb + k) & 1023) >> 7) << 7, _RUN)
                    pltpu.async_copy(outt_hbm.at[t].at[pl.ds(rb, _RUN)],
                                     run_v.at[i * _L + k], sem)
                return carry
            lax.fori_loop(0, _CH // _L, fire_body, 0)

        def drain_and_acc(g, acc_outer):
            e0 = g * _CH
            run_v, sem = bufs[g % 2], sems[g % 2]
            pltpu.make_async_copy(
                outt_hbm.at[pl.ds(0, _CH)].at[:, pl.ds(0, _RUN)],
                run_v, sem).wait()

            def acc_body(k, acc):
                off = e0 + k * _L
                pos = base_elem + off + lane
                v = plsc.load_gather(run_v, [k * _L + lane, pos & 127])
                return acc + v * wv_v[pl.ds(off, _L)]
            return lax.fori_loop(0, _CH // _L, acc_body, acc_outer)

        for cp in wcopies:
            cp.wait()

        acc = jnp.zeros((_L,), jnp.float32)
        fire(0)
        for g in range(1, _NCHUNK):
            fire(g)
            acc = drain_and_acc(g - 1, acc)
        acc = drain_and_acc(_NCHUNK - 1, acc)
        acc_v[...] = acc
        pltpu.sync_copy(acc_v, part_hbm.at[wid])

    return bow_partials


_bow_partials = _make_sc_call()


def kernel(output, target, weight):
    tgt_flat = target.T.reshape(-1)
    partials = _bow_partials(output.T, tgt_flat, weight)
    return -(jnp.sum(partials) / _BATCH)


# speedup vs baseline: 21.7774x; 1.0005x over previous
import functools

import jax
import jax.numpy as jnp
from jax import lax
from jax.experimental import pallas as pl
from jax.experimental.pallas import tpu as pltpu
from jax.experimental.pallas import tpu_sc as plsc

_BATCH = 1024
_TGT = 50
_VOCAB = 100000
_NC = 2
_NS = 16
_L = 16
_NW = _NC * _NS
_ROWS_W = _BATCH // _NW
_E = _ROWS_W * _TGT
_RUN = 128
_CH = 160
_NCHUNK = _E // _CH
_WCH = 128
_NWCH = _E // _WCH
_WREM = _E - _NWCH * _WCH


def _make_sc_call():
    mesh = plsc.VectorSubcoreMesh(
        core_axis_name="c", subcore_axis_name="s",
        num_cores=_NC, num_subcores=_NS)

    @functools.partial(
        pl.kernel,
        out_type=jax.ShapeDtypeStruct((_NW, _L), jnp.float32),
        mesh=mesh,
        compiler_params=pltpu.CompilerParams(needs_layout_passes=False, skip_device_barrier=True),
        scratch_types=[
            pltpu.VMEM((_E,), jnp.int32),
            pltpu.VMEM((_CH, _RUN), jnp.float32),
            pltpu.VMEM((_CH, _RUN), jnp.float32),
            pltpu.VMEM((_E,), jnp.float32),
            pltpu.VMEM((_L,), jnp.float32),
            pltpu.SemaphoreType.DMA,
            pltpu.SemaphoreType.DMA,
            pltpu.SemaphoreType.DMA,
        ],
    )
    def bow_partials(outt_hbm, tgt_hbm, wt_hbm, part_hbm,
                     tgt_v, run_a, run_b, wv_v, acc_v, sem_a, sem_b, wsem):
        cid = lax.axis_index("c")
        sid = lax.axis_index("s")
        wid = sid * _NC + cid
        base_elem = wid * _E
        lane = lax.iota(jnp.int32, _L)

        pltpu.sync_copy(tgt_hbm.at[pl.ds(base_elem, _E)], tgt_v)

        wcopies = []
        for c in range(_NWCH + 1):
            off = c * _WCH
            sz = _WCH if c < _NWCH else _WREM
            wcopies.append(pltpu.async_copy(
                wt_hbm.at[tgt_v.at[pl.ds(off, sz)]],
                wv_v.at[pl.ds(off, sz)], wsem))

        izeros = jnp.zeros((_L,), jnp.int32)
        bufs = [run_a, run_b]
        sems = [sem_a, sem_b]

        def fire(g):
            e0 = g * _CH
            run_v, sem = bufs[g % 2], sems[g % 2]

            def fire_body(i, carry):
                eb = e0 + i * _L
                chunk16 = tgt_v[pl.ds(eb, _L)]
                for k in range(_L):
                    t = chunk16[k]
                    rb = pl.multiple_of(
                        (((base_elem + eb + k) & 1023) >> 7) << 7, _RUN)
                    pltpu.async_copy(outt_hbm.at[t].at[pl.ds(rb, _RUN)],
                                     run_v.at[i * _L + k], sem)
                return carry
            lax.fori_loop(0, _CH // _L, fire_body, 0)

        def drain_and_acc(g, acc_outer):
            e0 = g * _CH
            run_v, sem = bufs[g % 2], sems[g % 2]
            pltpu.make_async_copy(
                outt_hbm.at[pl.ds(0, _CH)].at[:, pl.ds(0, _RUN)],
                run_v, sem).wait()

            def acc_body(k, acc):
                off = e0 + k * _L
                pos = base_elem + off + lane
                v = plsc.load_gather(run_v, [k * _L + lane, pos & 127])
                return acc + v * wv_v[pl.ds(off, _L)]
            return lax.fori_loop(0, _CH // _L, acc_body, acc_outer)

        for cp in wcopies:
            cp.wait()

        acc = jnp.zeros((_L,), jnp.float32)
        fire(0)
        for g in range(1, _NCHUNK):
            fire(g)
            acc = drain_and_acc(g - 1, acc)
        acc = drain_and_acc(_NCHUNK - 1, acc)
        acc_v[...] = acc
        pltpu.sync_copy(acc_v, part_hbm.at[wid])

    return bow_partials


_bow_partials = _make_sc_call()


def kernel(output, target, weight):
    tgt_flat = target.T.reshape(-1)
    partials = _bow_partials(output.T, tgt_flat, weight)
    return -(jnp.sum(partials) / _BATCH)


# speedup vs baseline: 21.8391x; 1.0028x over previous
import functools

import jax
import jax.numpy as jnp
from jax import lax
from jax.experimental import pallas as pl
from jax.experimental.pallas import tpu as pltpu
from jax.experimental.pallas import tpu_sc as plsc

_BATCH = 1024
_TGT = 50
_VOCAB = 100000
_NC = 2
_NS = 16
_L = 16
_NW = _NC * _NS
_ROWS_W = _BATCH // _NW
_E = _ROWS_W * _TGT
_RUN = 128
_CH = 160
_NCHUNK = _E // _CH
_WCH = 128
_NWCH = _E // _WCH
_WREM = _E - _NWCH * _WCH


def _make_sc_call():
    mesh = plsc.VectorSubcoreMesh(
        core_axis_name="c", subcore_axis_name="s",
        num_cores=_NC, num_subcores=_NS)

    @functools.partial(
        pl.kernel,
        out_type=jax.ShapeDtypeStruct((_NW, _L), jnp.float32),
        mesh=mesh,
        compiler_params=pltpu.CompilerParams(needs_layout_passes=False),
        scratch_types=[
            pltpu.VMEM((_E,), jnp.int32),
            pltpu.VMEM((_CH, _RUN), jnp.float32),
            pltpu.VMEM((_CH, _RUN), jnp.float32),
            pltpu.VMEM((_E,), jnp.float32),
            pltpu.VMEM((_L,), jnp.float32),
            pltpu.SemaphoreType.DMA,
            pltpu.SemaphoreType.DMA,
            pltpu.SemaphoreType.DMA,
        ],
    )
    def bow_partials(outt_hbm, tgt_hbm, wt_hbm, part_hbm,
                     tgt_v, run_a, run_b, wv_v, acc_v, sem_a, sem_b, wsem):
        cid = lax.axis_index("c")
        sid = lax.axis_index("s")
        wid = sid * _NC + cid
        base_elem = wid * _E
        lane = lax.iota(jnp.int32, _L)

        pltpu.sync_copy(tgt_hbm.at[pl.ds(base_elem, _E)], tgt_v)

        wcopies = []
        for c in range(_NWCH + 1):
            off = c * _WCH
            sz = _WCH if c < _NWCH else _WREM
            wcopies.append(pltpu.async_copy(
                wt_hbm.at[tgt_v.at[pl.ds(off, sz)]],
                wv_v.at[pl.ds(off, sz)], wsem))

        izeros = jnp.zeros((_L,), jnp.int32)
        bufs = [run_a, run_b]
        sems = [sem_a, sem_b]

        def fire(g):
            e0 = g * _CH
            run_v, sem = bufs[g % 2], sems[g % 2]

            def fire_body(i, carry):
                eb = e0 + i * _L
                chunk16 = tgt_v[pl.ds(eb, _L)]
                for k in range(_L):
                    t = chunk16[k]
                    rb = pl.multiple_of(
                        (((base_elem + eb + k) & 1023) >> 7) << 7, _RUN)
                    pltpu.async_copy(outt_hbm.at[t].at[pl.ds(rb, _RUN)],
                                     run_v.at[i * _L + k], sem)
                return carry
            lax.fori_loop(0, _CH // _L, fire_body, 0)

        def drain_and_acc(g, acc_outer):
            e0 = g * _CH
            run_v, sem = bufs[g % 2], sems[g % 2]
            pltpu.make_async_copy(
                outt_hbm.at[pl.ds(0, _CH)].at[:, pl.ds(0, _RUN)],
                run_v, sem).wait()

            def acc_body(k, acc):
                off = e0 + k * _L
                pos = base_elem + off + lane
                v = plsc.load_gather(run_v, [k * _L + lane, pos & 127])
                return acc + v * wv_v[pl.ds(off, _L)]
            return lax.fori_loop(0, _CH // _L, acc_body, acc_outer)

        acc = jnp.zeros((_L,), jnp.float32)
        fire(0)
        for g in range(1, _NCHUNK):
            fire(g)
            if g == 1:
                for cp in wcopies:
                    cp.wait()
            acc = drain_and_acc(g - 1, acc)
        acc = drain_and_acc(_NCHUNK - 1, acc)
        acc_v[...] = acc
        pltpu.sync_copy(acc_v, part_hbm.at[wid])

    return bow_partials


_bow_partials = _make_sc_call()


def kernel(output, target, weight):
    tgt_flat = target.T.reshape(-1)
    partials = _bow_partials(output.T, tgt_flat, weight)
    return -(jnp.sum(partials) / _BATCH)
